# baseline (speedup 1.0000x reference)
import functools

import jax
import jax.numpy as jnp
from jax import lax
from jax.experimental import pallas as pl
from jax.experimental.pallas import tpu as pltpu
from jax.experimental.pallas import tpu_sc as plsc

N_IN = 10242
N_OUT = 2562
K = 7
C = 128
BT = 64

NC_NODES = 16
M = NC_NODES * K
NUM_CHUNKS = -(-N_OUT // NC_NODES)
N_PAD = NUM_CHUNKS * NC_NODES
NW = 32
CHUNKS_PER_W = -(-NUM_CHUNKS // NW)

_mesh = plsc.VectorSubcoreMesh(core_axis_name="c", subcore_axis_name="s")


@functools.partial(
    pl.kernel,
    out_type=jax.ShapeDtypeStruct((BT, N_PAD, C), jnp.float32),
    mesh=_mesh,
    scratch_types=[
        pltpu.VMEM((M,), jnp.int32),
        pltpu.VMEM((M,), jnp.int32),
        pltpu.VMEM((M, C), jnp.float32),
        pltpu.VMEM((NC_NODES, C), jnp.float32),
        pltpu.SemaphoreType.DMA,
    ],
)
def _sphere_pool_sc(tensor_hbm, idxp_hbm, out_hbm, idx_base, idx_v, rows_v,
                    out_v, sem):
    cid = lax.axis_index("c")
    sid = lax.axis_index("s")
    wid = sid * 2 + cid

    for i in range(CHUNKS_PER_W):
        j = wid + i * NW

        @pl.when(j < NUM_CHUNKS)
        def _():
            start = pl.multiple_of(j * NC_NODES, NC_NODES)
            pltpu.sync_copy(idxp_hbm.at[j], idx_base)

            def bt_body(bt, carry):
                off = bt * N_IN
                for t in range(M // 16):
                    sl = pl.ds(t * 16, 16)
                    idx_v[sl] = idx_base[sl] + off
                pltpu.async_copy(tensor_hbm.at[idx_v], rows_v, sem).wait()

                def node_body(n, c2):
                    b = n * K
                    for s8 in range(C // 16):
                        sl = pl.ds(s8 * 16, 16)
                        m = rows_v[b, sl]
                        for k in range(1, K):
                            m = jnp.maximum(m, rows_v[b + k, sl])
                        out_v[n, sl] = m
                    return c2

                lax.fori_loop(0, NC_NODES, node_body, 0)
                pltpu.sync_copy(out_v,
                                out_hbm.at[bt, pl.ds(start, NC_NODES)])
                return carry

            lax.fori_loop(0, BT, bt_body, 0)


def kernel(tensor, index):
    B, T, N, Ch = tensor.shape
    x = tensor.reshape(B * T * N, Ch)
    starts = jnp.arange(NUM_CHUNKS, dtype=jnp.int32) * NC_NODES
    rows = starts[:, None] + jnp.arange(NC_NODES, dtype=jnp.int32)[None, :]
    idxp = index[rows].reshape(NUM_CHUNKS, M)
    out = _sphere_pool_sc(x, idxp)
    return out[:, :N_OUT].reshape(B, T, N_OUT, Ch)


# speedup vs baseline: 1.4364x; 1.4364x over previous
import functools

import jax
import jax.numpy as jnp
from jax import lax
from jax.experimental import pallas as pl
from jax.experimental.pallas import tpu as pltpu
from jax.experimental.pallas import tpu_sc as plsc

N_IN = 10242
N_OUT = 2562
K = 7
C = 128
BT = 64

NC_NODES = 16
M = NC_NODES * K
NUM_CHUNKS = -(-N_OUT // NC_NODES)
N_PAD = NUM_CHUNKS * NC_NODES
NW = 32
CHUNKS_PER_W = -(-NUM_CHUNKS // NW)

_mesh = plsc.VectorSubcoreMesh(core_axis_name="c", subcore_axis_name="s")


@functools.partial(
    pl.kernel,
    out_type=jax.ShapeDtypeStruct((BT, N_PAD, C), jnp.float32),
    mesh=_mesh,
    scratch_types=[
        pltpu.VMEM((M,), jnp.int32),
        pltpu.VMEM((M,), jnp.int32),
        pltpu.VMEM((M,), jnp.int32),
        pltpu.VMEM((M, C), jnp.float32),
        pltpu.VMEM((M, C), jnp.float32),
        pltpu.VMEM((NC_NODES, C), jnp.float32),
        pltpu.VMEM((NC_NODES, C), jnp.float32),
        pltpu.SemaphoreType.DMA,
        pltpu.SemaphoreType.DMA,
        pltpu.SemaphoreType.DMA,
        pltpu.SemaphoreType.DMA,
    ],
)
def _sphere_pool_sc(tensor_hbm, idxp_hbm, out_hbm, idx_base, iv0, iv1,
                    rows0, rows1, out0, out1, sg0, sg1, sw0, sw1):
    cid = lax.axis_index("c")
    sid = lax.axis_index("s")
    wid = sid * 2 + cid

    def set_idx(iv, bt):
        off = bt * N_IN
        for t in range(M // 16):
            sl = pl.ds(t * 16, 16)
            iv[sl] = idx_base[sl] + off

    def reduce_chunk(rows_v, out_v):
        def node_body(n, c2):
            b = n * K
            for s8 in range(C // 16):
                sl = pl.ds(s8 * 16, 16)
                m = rows_v[b, sl]
                for k in range(1, K):
                    m = jnp.maximum(m, rows_v[b + k, sl])
                out_v[n, sl] = m
            return c2

        lax.fori_loop(0, NC_NODES, node_body, 0)

    for i in range(CHUNKS_PER_W):
        j = wid + i * NW

        @pl.when(j < NUM_CHUNKS)
        def _():
            start = pl.multiple_of(j * NC_NODES, NC_NODES)
            pltpu.sync_copy(idxp_hbm.at[j], idx_base)

            set_idx(iv0, 0)
            pltpu.async_copy(tensor_hbm.at[iv0], rows0, sg0)
            set_idx(iv1, 1)
            pltpu.async_copy(tensor_hbm.at[iv1], rows1, sg1)

            def half_body(h, carry):
                bt0 = 2 * h

                pltpu.make_async_copy(tensor_hbm.at[iv0], rows0, sg0).wait()

                @pl.when(h > 0)
                def _():
                    pltpu.make_async_copy(
                        out0, out_hbm.at[bt0, pl.ds(start, NC_NODES)],
                        sw0).wait()

                reduce_chunk(rows0, out0)
                pltpu.async_copy(
                    out0, out_hbm.at[bt0, pl.ds(start, NC_NODES)], sw0)

                @pl.when(h < BT // 2 - 1)
                def _():
                    set_idx(iv0, bt0 + 2)
                    pltpu.async_copy(tensor_hbm.at[iv0], rows0, sg0)

                pltpu.make_async_copy(tensor_hbm.at[iv1], rows1, sg1).wait()

                @pl.when(h > 0)
                def _():
                    pltpu.make_async_copy(
                        out1, out_hbm.at[bt0 + 1, pl.ds(start, NC_NODES)],
                        sw1).wait()

                reduce_chunk(rows1, out1)
                pltpu.async_copy(
                    out1, out_hbm.at[bt0 + 1, pl.ds(start, NC_NODES)], sw1)

                @pl.when(h < BT // 2 - 1)
                def _():
                    set_idx(iv1, bt0 + 3)
                    pltpu.async_copy(tensor_hbm.at[iv1], rows1, sg1)

                return carry

            lax.fori_loop(0, BT // 2, half_body, 0)

            pltpu.make_async_copy(
                out0, out_hbm.at[0, pl.ds(start, NC_NODES)], sw0).wait()
            pltpu.make_async_copy(
                out1, out_hbm.at[0, pl.ds(start, NC_NODES)], sw1).wait()


def kernel(tensor, index):
    B, T, N, Ch = tensor.shape
    x = tensor.reshape(B * T * N, Ch)
    starts = jnp.arange(NUM_CHUNKS, dtype=jnp.int32) * NC_NODES
    rows = starts[:, None] + jnp.arange(NC_NODES, dtype=jnp.int32)[None, :]
    idxp = index[rows].reshape(NUM_CHUNKS, M)
    out = _sphere_pool_sc(x, idxp)
    return out[:, :N_OUT].reshape(B, T, N_OUT, Ch)


# speedup vs baseline: 1.6722x; 1.1642x over previous
import functools

import jax
import jax.numpy as jnp
from jax import lax
from jax.experimental import pallas as pl
from jax.experimental.pallas import tpu as pltpu
from jax.experimental.pallas import tpu_sc as plsc

N_IN = 10242
N_OUT = 2562
K = 7
C = 128
BT = 64

NC_NODES = 16
M = NC_NODES * K
NUM_FULL = N_OUT // NC_NODES
TAIL_START = NUM_FULL * NC_NODES
TAIL_N = N_OUT - TAIL_START
NW = 32
CHUNKS_PER_W = NUM_FULL // NW

_mesh = plsc.VectorSubcoreMesh(core_axis_name="c", subcore_axis_name="s")


@functools.partial(
    pl.kernel,
    out_type=jax.ShapeDtypeStruct((BT, N_OUT, C), jnp.float32),
    mesh=_mesh,
    scratch_types=[
        pltpu.VMEM((M,), jnp.int32),
        pltpu.VMEM((M,), jnp.int32),
        pltpu.VMEM((M,), jnp.int32),
        pltpu.VMEM((M, C), jnp.float32),
        pltpu.VMEM((M, C), jnp.float32),
        pltpu.VMEM((NC_NODES, C), jnp.float32),
        pltpu.VMEM((NC_NODES, C), jnp.float32),
        pltpu.VMEM((1, M), jnp.int32),
        pltpu.VMEM((16,), jnp.int32),
        pltpu.VMEM((16,), jnp.int32),
        pltpu.VMEM((16, C), jnp.float32),
        pltpu.VMEM((16, C), jnp.float32),
        pltpu.VMEM((TAIL_N, C), jnp.float32),
        pltpu.VMEM((TAIL_N, C), jnp.float32),
        pltpu.SemaphoreType.DMA,
        pltpu.SemaphoreType.DMA,
        pltpu.SemaphoreType.DMA,
        pltpu.SemaphoreType.DMA,
    ],
)
def _sphere_pool_sc(tensor_hbm, idxp_hbm, out_hbm, idx_base, iv0, iv1,
                    rows0, rows1, out0, out1, tail_base, ivt0, ivt1,
                    rt0, rt1, ot0, ot1,
                    sg0, sg1, sw0, sw1):
    cid = lax.axis_index("c")
    sid = lax.axis_index("s")
    wid = sid * 2 + cid

    def set_idx(iv, bt):
        off = bt * N_IN
        for t in range(M // 16):
            sl = pl.ds(t * 16, 16)
            iv[sl] = idx_base[sl] + off

    def reduce_chunk(rows_v, out_v):
        def node_body(n, c2):
            b = n * K
            for s8 in range(C // 16):
                sl = pl.ds(s8 * 16, 16)
                m = rows_v[b, sl]
                for k in range(1, K):
                    m = jnp.maximum(m, rows_v[b + k, sl])
                out_v[n, sl] = m
            return c2

        lax.fori_loop(0, NC_NODES, node_body, 0)

    for i in range(CHUNKS_PER_W):
        j = wid + i * NW
        start = pl.multiple_of(j * NC_NODES, NC_NODES)
        pltpu.sync_copy(idxp_hbm.at[j], idx_base)

        set_idx(iv0, 0)
        pltpu.async_copy(tensor_hbm.at[iv0], rows0, sg0)
        set_idx(iv1, 1)
        pltpu.async_copy(tensor_hbm.at[iv1], rows1, sg1)

        def half_body(h, carry):
            bt0 = 2 * h

            pltpu.make_async_copy(tensor_hbm.at[iv0], rows0, sg0).wait()

            @pl.when(h > 0)
            def _():
                pltpu.make_async_copy(
                    out0, out_hbm.at[bt0, pl.ds(start, NC_NODES)],
                    sw0).wait()

            reduce_chunk(rows0, out0)
            pltpu.async_copy(
                out0, out_hbm.at[bt0, pl.ds(start, NC_NODES)], sw0)

            @pl.when(h < BT // 2 - 1)
            def _():
                set_idx(iv0, bt0 + 2)
                pltpu.async_copy(tensor_hbm.at[iv0], rows0, sg0)

            pltpu.make_async_copy(tensor_hbm.at[iv1], rows1, sg1).wait()

            @pl.when(h > 0)
            def _():
                pltpu.make_async_copy(
                    out1, out_hbm.at[bt0 + 1, pl.ds(start, NC_NODES)],
                    sw1).wait()

            reduce_chunk(rows1, out1)
            pltpu.async_copy(
                out1, out_hbm.at[bt0 + 1, pl.ds(start, NC_NODES)], sw1)

            @pl.when(h < BT // 2 - 1)
            def _():
                set_idx(iv1, bt0 + 3)
                pltpu.async_copy(tensor_hbm.at[iv1], rows1, sg1)

            return carry

        lax.fori_loop(0, BT // 2, half_body, 0)

        pltpu.make_async_copy(
            out0, out_hbm.at[0, pl.ds(start, NC_NODES)], sw0).wait()
        pltpu.make_async_copy(
            out1, out_hbm.at[0, pl.ds(start, NC_NODES)], sw1).wait()

    bt0 = 2 * wid
    pltpu.sync_copy(idxp_hbm.at[pl.ds(NUM_FULL, 1)], tail_base)
    ivt0[pl.ds(0, 16)] = tail_base[0, pl.ds(0, 16)] + bt0 * N_IN
    pltpu.async_copy(tensor_hbm.at[ivt0], rt0, sg0)
    ivt1[pl.ds(0, 16)] = tail_base[0, pl.ds(0, 16)] + (bt0 + 1) * N_IN
    pltpu.async_copy(tensor_hbm.at[ivt1], rt1, sg1)

    def reduce_tail(rt, ot):
        for n in range(TAIL_N):
            b = n * K
            for s8 in range(C // 16):
                sl = pl.ds(s8 * 16, 16)
                m = rt[b, sl]
                for k in range(1, K):
                    m = jnp.maximum(m, rt[b + k, sl])
                ot[n, sl] = m

    pltpu.make_async_copy(tensor_hbm.at[ivt0], rt0, sg0).wait()
    reduce_tail(rt0, ot0)
    pltpu.async_copy(ot0, out_hbm.at[bt0, pl.ds(TAIL_START, TAIL_N)], sw0)

    pltpu.make_async_copy(tensor_hbm.at[ivt1], rt1, sg1).wait()
    reduce_tail(rt1, ot1)
    pltpu.async_copy(ot1, out_hbm.at[bt0 + 1, pl.ds(TAIL_START, TAIL_N)], sw1)

    pltpu.make_async_copy(
        ot0, out_hbm.at[bt0, pl.ds(TAIL_START, TAIL_N)], sw0).wait()
    pltpu.make_async_copy(
        ot1, out_hbm.at[bt0 + 1, pl.ds(TAIL_START, TAIL_N)], sw1).wait()


def kernel(tensor, index):
    B, T, N, Ch = tensor.shape
    x = tensor.reshape(B * T * N, Ch)
    starts = jnp.arange(NUM_FULL + 1, dtype=jnp.int32) * NC_NODES
    rows = starts[:, None] + jnp.arange(NC_NODES, dtype=jnp.int32)[None, :]
    idxp = index[rows].reshape(NUM_FULL + 1, M)
    out = _sphere_pool_sc(x, idxp)
    return out.reshape(B, T, N_OUT, Ch)


# speedup vs baseline: 2.4411x; 1.4598x over previous
import functools

import jax
import jax.numpy as jnp
from jax import lax
from jax.experimental import pallas as pl
from jax.experimental.pallas import tpu as pltpu
from jax.experimental.pallas import tpu_sc as plsc

N_IN = 10242
N_OUT = 2562
K = 7
C = 128
BT = 64

NC_NODES = 16
M = NC_NODES * K
NUM_FULL = N_OUT // NC_NODES
TAIL_START = NUM_FULL * NC_NODES
TAIL_N = N_OUT - TAIL_START
NW = 32
CHUNKS_PER_W = NUM_FULL // NW

_mesh = plsc.VectorSubcoreMesh(core_axis_name="c", subcore_axis_name="s")


@functools.partial(
    pl.kernel,
    out_type=jax.ShapeDtypeStruct((BT, N_OUT, C), jnp.float32),
    mesh=_mesh,
    scratch_types=[
        pltpu.VMEM((M,), jnp.int32),
        pltpu.VMEM((M, C), jnp.float32),
        pltpu.VMEM((M, C), jnp.float32),
        pltpu.VMEM((NC_NODES, C), jnp.float32),
        pltpu.VMEM((NC_NODES, C), jnp.float32),
        pltpu.VMEM((1, M), jnp.int32),
        pltpu.VMEM((16,), jnp.int32),
        pltpu.VMEM((16, C), jnp.float32),
        pltpu.VMEM((16, C), jnp.float32),
        pltpu.VMEM((TAIL_N, C), jnp.float32),
        pltpu.VMEM((TAIL_N, C), jnp.float32),
        pltpu.SemaphoreType.DMA,
        pltpu.SemaphoreType.DMA,
        pltpu.SemaphoreType.DMA,
        pltpu.SemaphoreType.DMA,
    ],
)
def _sphere_pool_sc(tensor_hbm, idxp_hbm, out_hbm, idx_v,
                    rows0, rows1, out0, out1, tail_base, ivt,
                    rt0, rt1, ot0, ot1, sg0, sg1, sw0, sw1):
    cid = lax.axis_index("c")
    sid = lax.axis_index("s")
    wid = sid * 2 + cid

    def reduce_chunk(rows_v, out_v):
        @plsc.parallel_loop(0, NC_NODES, 1, unroll=2)
        def node_body(n):
            b = n * K
            for s8 in range(C // 16):
                sl = pl.ds(s8 * 16, 16)
                t0 = jnp.maximum(rows_v[b, sl], rows_v[b + 1, sl])
                t1 = jnp.maximum(rows_v[b + 2, sl], rows_v[b + 3, sl])
                t2 = jnp.maximum(rows_v[b + 4, sl], rows_v[b + 5, sl])
                out_v[n, sl] = jnp.maximum(
                    jnp.maximum(t0, t1),
                    jnp.maximum(t2, rows_v[b + 6, sl]))

    for i in range(CHUNKS_PER_W):
        j = wid + i * NW
        start = pl.multiple_of(j * NC_NODES, NC_NODES)
        pltpu.sync_copy(idxp_hbm.at[j], idx_v)

        pltpu.async_copy(tensor_hbm.at[0].at[idx_v], rows0, sg0)
        pltpu.async_copy(tensor_hbm.at[1].at[idx_v], rows1, sg1)

        def half_body(h, carry):
            bt0 = 2 * h

            pltpu.make_async_copy(tensor_hbm.at[bt0].at[idx_v], rows0,
                                  sg0).wait()

            @pl.when(h > 0)
            def _():
                pltpu.make_async_copy(
                    out0, out_hbm.at[bt0, pl.ds(start, NC_NODES)],
                    sw0).wait()

            reduce_chunk(rows0, out0)
            pltpu.async_copy(
                out0, out_hbm.at[bt0, pl.ds(start, NC_NODES)], sw0)

            @pl.when(h < BT // 2 - 1)
            def _():
                pltpu.async_copy(tensor_hbm.at[bt0 + 2].at[idx_v], rows0,
                                 sg0)

            pltpu.make_async_copy(tensor_hbm.at[bt0 + 1].at[idx_v], rows1,
                                  sg1).wait()

            @pl.when(h > 0)
            def _():
                pltpu.make_async_copy(
                    out1, out_hbm.at[bt0 + 1, pl.ds(start, NC_NODES)],
                    sw1).wait()

            reduce_chunk(rows1, out1)
            pltpu.async_copy(
                out1, out_hbm.at[bt0 + 1, pl.ds(start, NC_NODES)], sw1)

            @pl.when(h < BT // 2 - 1)
            def _():
                pltpu.async_copy(tensor_hbm.at[bt0 + 3].at[idx_v], rows1,
                                 sg1)

            return carry

        lax.fori_loop(0, BT // 2, half_body, 0)

        pltpu.make_async_copy(
            out0, out_hbm.at[0, pl.ds(start, NC_NODES)], sw0).wait()
        pltpu.make_async_copy(
            out1, out_hbm.at[0, pl.ds(start, NC_NODES)], sw1).wait()

    bt0 = 2 * wid
    pltpu.sync_copy(idxp_hbm.at[pl.ds(NUM_FULL, 1)], tail_base)
    ivt[pl.ds(0, 16)] = tail_base[0, pl.ds(0, 16)]
    pltpu.async_copy(tensor_hbm.at[bt0].at[ivt], rt0, sg0)
    pltpu.async_copy(tensor_hbm.at[bt0 + 1].at[ivt], rt1, sg1)

    def reduce_tail(rt, ot):
        for n in range(TAIL_N):
            b = n * K
            for s8 in range(C // 16):
                sl = pl.ds(s8 * 16, 16)
                t0 = jnp.maximum(rt[b, sl], rt[b + 1, sl])
                t1 = jnp.maximum(rt[b + 2, sl], rt[b + 3, sl])
                t2 = jnp.maximum(rt[b + 4, sl], rt[b + 5, sl])
                ot[n, sl] = jnp.maximum(jnp.maximum(t0, t1),
                                        jnp.maximum(t2, rt[b + 6, sl]))

    pltpu.make_async_copy(tensor_hbm.at[bt0].at[ivt], rt0, sg0).wait()
    reduce_tail(rt0, ot0)
    pltpu.async_copy(ot0, out_hbm.at[bt0, pl.ds(TAIL_START, TAIL_N)], sw0)

    pltpu.make_async_copy(tensor_hbm.at[bt0 + 1].at[ivt], rt1, sg1).wait()
    reduce_tail(rt1, ot1)
    pltpu.async_copy(ot1, out_hbm.at[bt0 + 1, pl.ds(TAIL_START, TAIL_N)], sw1)

    pltpu.make_async_copy(
        ot0, out_hbm.at[bt0, pl.ds(TAIL_START, TAIL_N)], sw0).wait()
    pltpu.make_async_copy(
        ot1, out_hbm.at[bt0 + 1, pl.ds(TAIL_START, TAIL_N)], sw1).wait()


def kernel(tensor, index):
    B, T, N, Ch = tensor.shape
    x = tensor.reshape(B * T, N, Ch)
    starts = jnp.arange(NUM_FULL + 1, dtype=jnp.int32) * NC_NODES
    rows = starts[:, None] + jnp.arange(NC_NODES, dtype=jnp.int32)[None, :]
    idxp = index[rows].reshape(NUM_FULL + 1, M)
    out = _sphere_pool_sc(x, idxp)
    return out.reshape(B, T, N_OUT, Ch)


# speedup vs baseline: 2.4426x; 1.0006x over previous
import functools

import jax
import jax.numpy as jnp
from jax import lax
from jax.experimental import pallas as pl
from jax.experimental.pallas import tpu as pltpu
from jax.experimental.pallas import tpu_sc as plsc

N_IN = 10242
N_OUT = 2562
K = 7
C = 128
BT = 64

NC_NODES = 16
M = NC_NODES * K
NUM_FULL = N_OUT // NC_NODES
TAIL_START = NUM_FULL * NC_NODES
TAIL_N = N_OUT - TAIL_START
NW = 32
CHUNKS_PER_W = NUM_FULL // NW

_mesh = plsc.VectorSubcoreMesh(core_axis_name="c", subcore_axis_name="s")


@functools.partial(
    pl.kernel,
    out_type=jax.ShapeDtypeStruct((BT, N_OUT, C), jnp.float32),
    mesh=_mesh,
    scratch_types=[
        pltpu.VMEM((M,), jnp.int32),
        pltpu.VMEM((M, C), jnp.float32),
        pltpu.VMEM((M, C), jnp.float32),
        pltpu.VMEM((NC_NODES, C), jnp.float32),
        pltpu.VMEM((NC_NODES, C), jnp.float32),
        pltpu.VMEM((1, M), jnp.int32),
        pltpu.VMEM((16,), jnp.int32),
        pltpu.VMEM((16, C), jnp.float32),
        pltpu.VMEM((16, C), jnp.float32),
        pltpu.VMEM((TAIL_N, C), jnp.float32),
        pltpu.VMEM((TAIL_N, C), jnp.float32),
        pltpu.SemaphoreType.DMA,
        pltpu.SemaphoreType.DMA,
        pltpu.SemaphoreType.DMA,
        pltpu.SemaphoreType.DMA,
    ],
    compiler_params=pltpu.CompilerParams(use_tc_tiling_on_sc=True),
)
def _sphere_pool_sc(tensor_hbm, idxp_hbm, out_hbm, idx_v,
                    rows0, rows1, out0, out1, tail_base, ivt,
                    rt0, rt1, ot0, ot1, sg0, sg1, sw0, sw1):
    cid = lax.axis_index("c")
    sid = lax.axis_index("s")
    wid = sid * 2 + cid

    def reduce_chunk(rows_v, out_v):
        @plsc.parallel_loop(0, NC_NODES, 1, unroll=2)
        def node_body(n):
            b = n * K
            for s8 in range(C // 16):
                sl = pl.ds(s8 * 16, 16)
                t0 = jnp.maximum(rows_v[b, sl], rows_v[b + 1, sl])
                t1 = jnp.maximum(rows_v[b + 2, sl], rows_v[b + 3, sl])
                t2 = jnp.maximum(rows_v[b + 4, sl], rows_v[b + 5, sl])
                out_v[n, sl] = jnp.maximum(
                    jnp.maximum(t0, t1),
                    jnp.maximum(t2, rows_v[b + 6, sl]))

    for i in range(CHUNKS_PER_W):
        j = wid + i * NW
        start = pl.multiple_of(j * NC_NODES, NC_NODES)
        pltpu.sync_copy(idxp_hbm.at[j], idx_v)

        pltpu.async_copy(tensor_hbm.at[0].at[idx_v], rows0, sg0)
        pltpu.async_copy(tensor_hbm.at[1].at[idx_v], rows1, sg1)

        def half_body(h, carry):
            bt0 = 2 * h

            pltpu.make_async_copy(tensor_hbm.at[bt0].at[idx_v], rows0,
                                  sg0).wait()

            @pl.when(h > 0)
            def _():
                pltpu.make_async_copy(
                    out0, out_hbm.at[bt0, pl.ds(start, NC_NODES)],
                    sw0).wait()

            reduce_chunk(rows0, out0)
            pltpu.async_copy(
                out0, out_hbm.at[bt0, pl.ds(start, NC_NODES)], sw0)

            @pl.when(h < BT // 2 - 1)
            def _():
                pltpu.async_copy(tensor_hbm.at[bt0 + 2].at[idx_v], rows0,
                                 sg0)

            pltpu.make_async_copy(tensor_hbm.at[bt0 + 1].at[idx_v], rows1,
                                  sg1).wait()

            @pl.when(h > 0)
            def _():
                pltpu.make_async_copy(
                    out1, out_hbm.at[bt0 + 1, pl.ds(start, NC_NODES)],
                    sw1).wait()

            reduce_chunk(rows1, out1)
            pltpu.async_copy(
                out1, out_hbm.at[bt0 + 1, pl.ds(start, NC_NODES)], sw1)

            @pl.when(h < BT // 2 - 1)
            def _():
                pltpu.async_copy(tensor_hbm.at[bt0 + 3].at[idx_v], rows1,
                                 sg1)

            return carry

        lax.fori_loop(0, BT // 2, half_body, 0)

        pltpu.make_async_copy(
            out0, out_hbm.at[0, pl.ds(start, NC_NODES)], sw0).wait()
        pltpu.make_async_copy(
            out1, out_hbm.at[0, pl.ds(start, NC_NODES)], sw1).wait()

    bt0 = 2 * wid
    pltpu.sync_copy(idxp_hbm.at[pl.ds(NUM_FULL, 1)], tail_base)
    ivt[pl.ds(0, 16)] = tail_base[0, pl.ds(0, 16)]
    pltpu.async_copy(tensor_hbm.at[bt0].at[ivt], rt0, sg0)
    pltpu.async_copy(tensor_hbm.at[bt0 + 1].at[ivt], rt1, sg1)

    def reduce_tail(rt, ot):
        for n in range(TAIL_N):
            b = n * K
            for s8 in range(C // 16):
                sl = pl.ds(s8 * 16, 16)
                t0 = jnp.maximum(rt[b, sl], rt[b + 1, sl])
                t1 = jnp.maximum(rt[b + 2, sl], rt[b + 3, sl])
                t2 = jnp.maximum(rt[b + 4, sl], rt[b + 5, sl])
                ot[n, sl] = jnp.maximum(jnp.maximum(t0, t1),
                                        jnp.maximum(t2, rt[b + 6, sl]))

    pltpu.make_async_copy(tensor_hbm.at[bt0].at[ivt], rt0, sg0).wait()
    reduce_tail(rt0, ot0)
    pltpu.async_copy(ot0, out_hbm.at[bt0, pl.ds(TAIL_START, TAIL_N)], sw0)

    pltpu.make_async_copy(tensor_hbm.at[bt0 + 1].at[ivt], rt1, sg1).wait()
    reduce_tail(rt1, ot1)
    pltpu.async_copy(ot1, out_hbm.at[bt0 + 1, pl.ds(TAIL_START, TAIL_N)], sw1)

    pltpu.make_async_copy(
        ot0, out_hbm.at[bt0, pl.ds(TAIL_START, TAIL_N)], sw0).wait()
    pltpu.make_async_copy(
        ot1, out_hbm.at[bt0 + 1, pl.ds(TAIL_START, TAIL_N)], sw1).wait()


def kernel(tensor, index):
    B, T, N, Ch = tensor.shape
    x = tensor.reshape(B * T, N, Ch)
    starts = jnp.arange(NUM_FULL + 1, dtype=jnp.int32) * NC_NODES
    rows = starts[:, None] + jnp.arange(NC_NODES, dtype=jnp.int32)[None, :]
    idxp = index[rows].reshape(NUM_FULL + 1, M)
    out = _sphere_pool_sc(x, idxp)
    return out.reshape(B, T, N_OUT, Ch)


# speedup vs baseline: 2.7270x; 1.1164x over previous
import functools

import jax
import jax.numpy as jnp
from jax import lax
from jax.experimental import pallas as pl
from jax.experimental.pallas import tpu as pltpu
from jax.experimental.pallas import tpu_sc as plsc

N_IN = 10242
N_OUT = 2562
K = 7
C = 128
BT = 64

NC_NODES = 16
M = NC_NODES * K
NUM_FULL = N_OUT // NC_NODES
TAIL_START = NUM_FULL * NC_NODES
TAIL_N = N_OUT - TAIL_START
NW = 32
CHUNKS_PER_W = NUM_FULL // NW

_mesh = plsc.VectorSubcoreMesh(core_axis_name="c", subcore_axis_name="s")


@functools.partial(
    pl.kernel,
    out_type=jax.ShapeDtypeStruct((BT, N_OUT, C), jnp.float32),
    mesh=_mesh,
    scratch_types=[
        pltpu.VMEM((M,), jnp.int32),
        [pltpu.VMEM((M, C), jnp.float32) for _ in range(4)],
        [pltpu.VMEM((NC_NODES, C), jnp.float32) for _ in range(4)],
        pltpu.VMEM((1, M), jnp.int32),
        pltpu.VMEM((16,), jnp.int32),
        pltpu.VMEM((16, C), jnp.float32),
        pltpu.VMEM((16, C), jnp.float32),
        pltpu.VMEM((TAIL_N, C), jnp.float32),
        pltpu.VMEM((TAIL_N, C), jnp.float32),
        [pltpu.SemaphoreType.DMA for _ in range(4)],
        [pltpu.SemaphoreType.DMA for _ in range(4)],
    ],
)
def _sphere_pool_sc(tensor_hbm, idxp_hbm, out_hbm, idx_v,
                    rows, outs, tail_base, ivt,
                    rt0, rt1, ot0, ot1, sg, sw):
    cid = lax.axis_index("c")
    sid = lax.axis_index("s")
    wid = sid * 2 + cid

    def reduce_chunk(rows_v, out_v):
        @plsc.parallel_loop(0, NC_NODES, 1, unroll=2)
        def node_body(n):
            b = n * K
            for s8 in range(C // 16):
                sl = pl.ds(s8 * 16, 16)
                t0 = jnp.maximum(rows_v[b, sl], rows_v[b + 1, sl])
                t1 = jnp.maximum(rows_v[b + 2, sl], rows_v[b + 3, sl])
                t2 = jnp.maximum(rows_v[b + 4, sl], rows_v[b + 5, sl])
                out_v[n, sl] = jnp.maximum(
                    jnp.maximum(t0, t1),
                    jnp.maximum(t2, rows_v[b + 6, sl]))

    for i in range(CHUNKS_PER_W):
        j = wid + i * NW
        start = pl.multiple_of(j * NC_NODES, NC_NODES)
        pltpu.sync_copy(idxp_hbm.at[j], idx_v)

        for p in range(3):
            pltpu.async_copy(tensor_hbm.at[p].at[idx_v], rows[p], sg[p])

        def quad_body(q, carry):
            bt0 = 4 * q
            for p in range(4):
                bt = bt0 + p
                pltpu.make_async_copy(tensor_hbm.at[bt].at[idx_v], rows[p],
                                      sg[p]).wait()

                @pl.when(q > 0)
                def _():
                    pltpu.make_async_copy(
                        outs[p], out_hbm.at[bt, pl.ds(start, NC_NODES)],
                        sw[p]).wait()

                reduce_chunk(rows[p], outs[p])
                pltpu.async_copy(
                    outs[p], out_hbm.at[bt, pl.ds(start, NC_NODES)], sw[p])

                pn = (p + 3) % 4

                @pl.when(bt + 3 < BT)
                def _():
                    pltpu.async_copy(tensor_hbm.at[bt + 3].at[idx_v],
                                     rows[pn], sg[pn])

            return carry

        lax.fori_loop(0, BT // 4, quad_body, 0)

        for p in range(4):
            pltpu.make_async_copy(
                outs[p], out_hbm.at[0, pl.ds(start, NC_NODES)], sw[p]).wait()

    bt0 = 2 * wid
    pltpu.sync_copy(idxp_hbm.at[pl.ds(NUM_FULL, 1)], tail_base)
    ivt[pl.ds(0, 16)] = tail_base[0, pl.ds(0, 16)]
    pltpu.async_copy(tensor_hbm.at[bt0].at[ivt], rt0, sg[0])
    pltpu.async_copy(tensor_hbm.at[bt0 + 1].at[ivt], rt1, sg[1])

    def reduce_tail(rt, ot):
        for n in range(TAIL_N):
            b = n * K
            for s8 in range(C // 16):
                sl = pl.ds(s8 * 16, 16)
                t0 = jnp.maximum(rt[b, sl], rt[b + 1, sl])
                t1 = jnp.maximum(rt[b + 2, sl], rt[b + 3, sl])
                t2 = jnp.maximum(rt[b + 4, sl], rt[b + 5, sl])
                ot[n, sl] = jnp.maximum(jnp.maximum(t0, t1),
                                        jnp.maximum(t2, rt[b + 6, sl]))

    pltpu.make_async_copy(tensor_hbm.at[bt0].at[ivt], rt0, sg[0]).wait()
    reduce_tail(rt0, ot0)
    pltpu.async_copy(ot0, out_hbm.at[bt0, pl.ds(TAIL_START, TAIL_N)], sw[0])

    pltpu.make_async_copy(tensor_hbm.at[bt0 + 1].at[ivt], rt1, sg[1]).wait()
    reduce_tail(rt1, ot1)
    pltpu.async_copy(ot1, out_hbm.at[bt0 + 1, pl.ds(TAIL_START, TAIL_N)], sw[1])

    pltpu.make_async_copy(
        ot0, out_hbm.at[bt0, pl.ds(TAIL_START, TAIL_N)], sw[0]).wait()
    pltpu.make_async_copy(
        ot1, out_hbm.at[bt0 + 1, pl.ds(TAIL_START, TAIL_N)], sw[1]).wait()


def kernel(tensor, index):
    B, T, N, Ch = tensor.shape
    x = tensor.reshape(B * T, N, Ch)
    starts = jnp.arange(NUM_FULL + 1, dtype=jnp.int32) * NC_NODES
    rows = starts[:, None] + jnp.arange(NC_NODES, dtype=jnp.int32)[None, :]
    idxp = index[rows].reshape(NUM_FULL + 1, M)
    out = _sphere_pool_sc(x, idxp)
    return out.reshape(B, T, N_OUT, Ch)
